# Initial kernel scaffold; baseline (speedup 1.0000x reference)
import jax
import jax.numpy as jnp
from jax.experimental import pallas as pl


def kernel(unique_nodes_list, samp_neighs, features):
    raise NotImplementedError("write your pallas kernel here")


# speedup vs baseline: 1.2329x; 1.2329x over previous
import functools

import jax
import jax.numpy as jnp
from jax import lax
from jax.experimental import pallas as pl
from jax.experimental.pallas import tpu as pltpu, tpu_sc as plsc

N = 10000
K = 16
D = 256

NC = 2
NS = 16
NW = NC * NS
L = 16

G = 8
CH = 320
NP = NW * CH
NGRP = CH // G
NODE_GROUPS = NP // G


def _max_agg_body(idx_hbm, feat_hbm, out_hbm, idx_v, rows_v, out_v, sem):
    wid = lax.axis_index("s") * NC + lax.axis_index("c")
    gbase = wid * NGRP

    pltpu.sync_copy(idx_hbm.at[pl.ds(gbase, NGRP)], idx_v)

    def group_body(g, _):
        pltpu.async_copy(feat_hbm.at[idx_v.at[g]], rows_v, sem).wait()

        def node_body(j, _):
            rbase = j * K
            for c in range(D // L):
                col = pl.ds(c * L, L)
                acc = rows_v[rbase, col]
                for r in range(1, K):
                    acc = jnp.maximum(acc, rows_v[rbase + r, col])
                out_v[j, col] = acc
            return 0

        lax.fori_loop(0, G, node_body, 0)
        pltpu.sync_copy(out_v, out_hbm.at[pl.ds((gbase + g) * G, G)])
        return 0

    lax.fori_loop(0, NGRP, group_body, 0)


@functools.partial(jax.jit, static_argnums=())
def kernel(unique_nodes_list, samp_neighs, features):
    del unique_nodes_list
    idx = samp_neighs.astype(jnp.int32)
    idx = jnp.pad(idx, ((0, NP - N), (0, 0))).reshape(NODE_GROUPS, G * K)
    feats = features.astype(jnp.float32)

    run = pl.kernel(
        _max_agg_body,
        out_type=jax.ShapeDtypeStruct((NP, D), jnp.float32),
        mesh=plsc.VectorSubcoreMesh(core_axis_name="c", subcore_axis_name="s"),
        scratch_types=[
            pltpu.VMEM((NGRP, G * K), jnp.int32),
            pltpu.VMEM((G * K, D), jnp.float32),
            pltpu.VMEM((G, D), jnp.float32),
            pltpu.SemaphoreType.DMA,
        ],
    )
    return run(idx, feats)[:N]


# speedup vs baseline: 1.4624x; 1.1861x over previous
import functools

import jax
import jax.numpy as jnp
from jax import lax
from jax.experimental import pallas as pl
from jax.experimental.pallas import tpu as pltpu, tpu_sc as plsc

N = 10000
K = 16
D = 256

NC = 2
NS = 16
NW = NC * NS
L = 16

G = 8
CH = 320
NP = NW * CH
NGRP = CH // G
NODE_GROUPS = NP // G


def _max_agg_body(idx_hbm, feat_hbm, out_hbm, idx_v, rows0, rows1, out_v,
                  sem0, sem1):
    wid = lax.axis_index("s") * NC + lax.axis_index("c")
    gbase = wid * NGRP
    rows = (rows0, rows1)
    sems = (sem0, sem1)

    pltpu.sync_copy(idx_hbm.at[pl.ds(gbase, NGRP)], idx_v)

    pltpu.async_copy(feat_hbm.at[idx_v.at[0]], rows0, sem0)
    pltpu.async_copy(feat_hbm.at[idx_v.at[1]], rows1, sem1)

    def pair_body(p, _):
        for b in range(2):
            g = p * 2 + b
            rb, sb = rows[b], sems[b]
            pltpu.make_async_copy(feat_hbm.at[idx_v.at[g]], rb, sb).wait()

            def node_body(j, _):
                rbase = j * K
                for c in range(D // L):
                    col = pl.ds(c * L, L)
                    acc = rb[rbase, col]
                    for r in range(1, K):
                        acc = jnp.maximum(acc, rb[rbase + r, col])
                    out_v[j, col] = acc
                return 0

            lax.fori_loop(0, G, node_body, 0)
            pltpu.sync_copy(out_v, out_hbm.at[pl.ds((gbase + g) * G, G)])

            @pl.when(g + 2 < NGRP)
            def _():
                pltpu.async_copy(feat_hbm.at[idx_v.at[g + 2]], rb, sb)
        return 0

    lax.fori_loop(0, NGRP // 2, pair_body, 0)


@functools.partial(jax.jit, static_argnums=())
def kernel(unique_nodes_list, samp_neighs, features):
    del unique_nodes_list
    idx = samp_neighs.astype(jnp.int32)
    idx = jnp.pad(idx, ((0, NP - N), (0, 0))).reshape(NODE_GROUPS, G * K)
    feats = features.astype(jnp.float32)

    run = pl.kernel(
        _max_agg_body,
        out_type=jax.ShapeDtypeStruct((NP, D), jnp.float32),
        mesh=plsc.VectorSubcoreMesh(core_axis_name="c", subcore_axis_name="s"),
        scratch_types=[
            pltpu.VMEM((NGRP, G * K), jnp.int32),
            pltpu.VMEM((G * K, D), jnp.float32),
            pltpu.VMEM((G * K, D), jnp.float32),
            pltpu.VMEM((G, D), jnp.float32),
            pltpu.SemaphoreType.DMA,
            pltpu.SemaphoreType.DMA,
        ],
    )
    return run(idx, feats)[:N]
